# Initial kernel scaffold; baseline (speedup 1.0000x reference)
import jax
import jax.numpy as jnp
from jax.experimental import pallas as pl


def kernel(x, edge_index, edge_type, weights, weights_inv):
    raise NotImplementedError("write your pallas kernel here")


# speedup vs baseline: 11.0978x; 11.0978x over previous
import functools

import jax
import jax.numpy as jnp
from jax import lax
from jax.experimental import pallas as pl
from jax.experimental.pallas import tpu as pltpu
from jax.experimental.pallas import tpu_sc as plsc

_NC = 2
_NS = 16
_NW = _NC * _NS
_L = 16
_C = 128
_D = 2 * _C

_GATHER_DNUMS = lax.GatherDimensionNumbers(
    offset_dims=(), collapsed_slice_dims=(0,), start_index_map=(0,))


def _shuffle(v, idx):
    return lax.gather(v, idx[:, None], _GATHER_DNUMS, (1,),
                      mode=lax.GatherScatterMode.PROMISE_IN_BOUNDS)


def _hsum(v, lanes):
    for k in (8, 4, 2, 1):
        v = v + _shuffle(v, lanes ^ k)
    return v


def _sc_body(B, n_chunks, x2, wcat, src, dst, et, out,
             idx_s, idx_d, idx_t, rows_s, rows_d, rows_w, out_v,
             sem_s, sem_d, sem_w):
    epw = n_chunks * B
    wid = lax.axis_index("s") * _NC + lax.axis_index("c")
    base = wid * epw
    lanes = lax.iota(jnp.int32, _L)

    def chunk(c, carry):
        off = base + c * B
        pltpu.sync_copy(src.at[pl.ds(off, B)], idx_s)
        pltpu.sync_copy(dst.at[pl.ds(off, B)], idx_d)
        pltpu.sync_copy(et.at[pl.ds(off, B)], idx_t)
        cs = pltpu.async_copy(x2.at[idx_s], rows_s, sem_s)
        cd = pltpu.async_copy(x2.at[idx_d], rows_d, sem_d)
        cw = pltpu.async_copy(wcat.at[idx_t], rows_w, sem_w)
        cs.wait()
        cd.wait()
        cw.wait()

        def group(g, gcarry):
            ovec = jnp.zeros((_L,), jnp.float32)
            for e in range(_L):
                i = g * _L + e
                acc = jnp.zeros((_L,), jnp.float32)
                for j in range(_C // _L):
                    lo = j * _L
                    hi = _C + j * _L
                    acc = acc + (rows_s[i, pl.ds(lo, _L)]
                                 * rows_w[i, pl.ds(lo, _L)]
                                 * rows_d[i, pl.ds(hi, _L)])
                    acc = acc + (rows_d[i, pl.ds(lo, _L)]
                                 * rows_w[i, pl.ds(hi, _L)]
                                 * rows_s[i, pl.ds(hi, _L)])
                ovec = jnp.where(lanes == e, _hsum(acc, lanes), ovec)
            out_v[pl.ds(g * _L, _L)] = ovec * jnp.float32(0.5)
            return gcarry

        lax.fori_loop(0, B // _L, group, 0, unroll=False)
        pltpu.sync_copy(out_v, out.at[pl.ds(off, B)])
        return carry

    lax.fori_loop(0, n_chunks, chunk, 0, unroll=False)


@functools.partial(jax.jit, static_argnames=("B",))
def _simple_scores(x2, wcat, src, dst, et, B=80):
    E = src.shape[0]
    assert E % (_NW * B) == 0 and B % 8 == 0
    n_chunks = E // (_NW * B)
    mesh = plsc.VectorSubcoreMesh(core_axis_name="c", subcore_axis_name="s")
    body = functools.partial(_sc_body, B, n_chunks)
    return pl.kernel(
        body,
        out_type=jax.ShapeDtypeStruct((E,), jnp.float32),
        mesh=mesh,
        scratch_types=[
            pltpu.VMEM((B,), jnp.int32),
            pltpu.VMEM((B,), jnp.int32),
            pltpu.VMEM((B,), jnp.int32),
            pltpu.VMEM((B, _D), jnp.float32),
            pltpu.VMEM((B, _D), jnp.float32),
            pltpu.VMEM((B, _D), jnp.float32),
            pltpu.VMEM((B,), jnp.float32),
            pltpu.SemaphoreType.DMA,
            pltpu.SemaphoreType.DMA,
            pltpu.SemaphoreType.DMA,
        ],
    )(x2, wcat, src, dst, et)


def kernel(x, edge_index, edge_type, weights, weights_inv):
    n = x.shape[0]
    x2 = x.reshape(n, _D)
    wcat = jnp.concatenate([weights, weights_inv], axis=1)
    return _simple_scores(x2, wcat, edge_index[0], edge_index[1], edge_type)


# speedup vs baseline: 45.2325x; 4.0758x over previous
import functools

import jax
import jax.numpy as jnp
from jax import lax
from jax.experimental import pallas as pl
from jax.experimental.pallas import tpu as pltpu
from jax.experimental.pallas import tpu_sc as plsc

_NC = 2
_NS = 16
_NW = _NC * _NS
_L = 16
_C = 128
_D = 2 * _C

_GATHER_DNUMS = lax.GatherDimensionNumbers(
    offset_dims=(), collapsed_slice_dims=(0,), start_index_map=(0,))


def _shuffle(v, idx):
    return lax.gather(v, idx[:, None], _GATHER_DNUMS, (1,),
                      mode=lax.GatherScatterMode.PROMISE_IN_BOUNDS)


def _hsum(v, lanes):
    for k in (8, 4, 2, 1):
        v = v + _shuffle(v, lanes ^ k)
    return v


def _sc_body(B, n_chunks, x2, wcat, eidx, out,
             idx_a, idx_b, rs_a, rd_a, rw_a, rs_b, rd_b, rw_b, ov_a, ov_b,
             semi_a, semi_b, semr_a, semr_b):
    wid = lax.axis_index("s") * _NC + lax.axis_index("c")
    ebase = wid * n_chunks * (3 * B)
    obase = wid * n_chunks * B
    lanes = lax.iota(jnp.int32, _L)

    def idx_desc(c, idxbuf, sem):
        return pltpu.make_async_copy(
            eidx.at[pl.ds(ebase + c * (3 * B), 3 * B)], idxbuf, sem)

    def row_descs(idxbuf, rs, rd, rw, sem):
        return (
            pltpu.make_async_copy(x2.at[idxbuf.at[pl.ds(0, B)]], rs, sem),
            pltpu.make_async_copy(x2.at[idxbuf.at[pl.ds(B, B)]], rd, sem),
            pltpu.make_async_copy(wcat.at[idxbuf.at[pl.ds(2 * B, B)]], rw, sem),
        )

    def fire_rows(idxbuf, rs, rd, rw, sem):
        for d in row_descs(idxbuf, rs, rd, rw, sem):
            d.start()

    def wait_rows(idxbuf, rs, rd, rw, sem):
        for d in row_descs(idxbuf, rs, rd, rw, sem):
            d.wait()

    def compute(rs, rd, rw, ov, c):
        def edge(e, ovec):
            acc = jnp.zeros((_L,), jnp.float32)
            for j in range(_C // _L):
                lo = j * _L
                hi = _C + j * _L
                acc = acc + (rs[e, pl.ds(lo, _L)]
                             * rw[e, pl.ds(lo, _L)]
                             * rd[e, pl.ds(hi, _L)])
                acc = acc + (rd[e, pl.ds(lo, _L)]
                             * rw[e, pl.ds(hi, _L)]
                             * rs[e, pl.ds(hi, _L)])
            lane = jnp.bitwise_and(e, _L - 1)
            ovec = jnp.where(lanes == lane, _hsum(acc, lanes), ovec)

            @pl.when(lane == _L - 1)
            def _():
                base = jnp.bitwise_and(e, ~(_L - 1))
                ov[pl.ds(base, _L)] = ovec * jnp.float32(0.5)

            return ovec

        lax.fori_loop(0, B, edge, jnp.zeros((_L,), jnp.float32),
                      unroll=False)
        pltpu.sync_copy(ov, out.at[pl.ds(obase + c * B, B)])

    idx_desc(0, idx_a, semi_a).start()
    idx_desc(1, idx_b, semi_b).start()
    idx_desc(0, idx_a, semi_a).wait()
    fire_rows(idx_a, rs_a, rd_a, rw_a, semr_a)

    def step(k, carry):
        c0 = 2 * k
        idx_desc(c0 + 1, idx_b, semi_b).wait()
        fire_rows(idx_b, rs_b, rd_b, rw_b, semr_b)
        wait_rows(idx_a, rs_a, rd_a, rw_a, semr_a)
        idx_desc(c0 + 2, idx_a, semi_a).start()
        compute(rs_a, rd_a, rw_a, ov_a, c0)
        idx_desc(c0 + 2, idx_a, semi_a).wait()
        fire_rows(idx_a, rs_a, rd_a, rw_a, semr_a)
        wait_rows(idx_b, rs_b, rd_b, rw_b, semr_b)

        @pl.when(k < (n_chunks - 3) // 2)
        def _():
            idx_desc(c0 + 3, idx_b, semi_b).start()

        compute(rs_b, rd_b, rw_b, ov_b, c0 + 1)
        return carry

    lax.fori_loop(0, (n_chunks - 1) // 2, step, 0, unroll=False)

    wait_rows(idx_a, rs_a, rd_a, rw_a, semr_a)
    compute(rs_a, rd_a, rw_a, ov_a, n_chunks - 1)


@functools.partial(jax.jit, static_argnames=("B",))
def _simple_scores(x2, wcat, eidx, B=80):
    E = eidx.shape[0] // 3
    assert E % (_NW * B) == 0 and B % _L == 0
    n_chunks = E // (_NW * B)
    assert n_chunks % 2 == 1
    mesh = plsc.VectorSubcoreMesh(core_axis_name="c", subcore_axis_name="s")
    body = functools.partial(_sc_body, B, n_chunks)
    return pl.kernel(
        body,
        out_type=jax.ShapeDtypeStruct((E,), jnp.float32),
        mesh=mesh,
        scratch_types=[
            pltpu.VMEM((3 * B,), jnp.int32),
            pltpu.VMEM((3 * B,), jnp.int32),
            pltpu.VMEM((B, _D), jnp.float32),
            pltpu.VMEM((B, _D), jnp.float32),
            pltpu.VMEM((B, _D), jnp.float32),
            pltpu.VMEM((B, _D), jnp.float32),
            pltpu.VMEM((B, _D), jnp.float32),
            pltpu.VMEM((B, _D), jnp.float32),
            pltpu.VMEM((B,), jnp.float32),
            pltpu.VMEM((B,), jnp.float32),
            pltpu.SemaphoreType.DMA,
            pltpu.SemaphoreType.DMA,
            pltpu.SemaphoreType.DMA,
            pltpu.SemaphoreType.DMA,
        ],
    )(x2, wcat, eidx)


def kernel(x, edge_index, edge_type, weights, weights_inv, B=80):
    n = x.shape[0]
    E = edge_type.shape[0]
    n_chunks = E // (_NW * B)
    x2 = x.reshape(n, _D)
    wcat = jnp.concatenate([weights, weights_inv], axis=1)
    trip = jnp.stack([
        edge_index[0].reshape(_NW, n_chunks, B),
        edge_index[1].reshape(_NW, n_chunks, B),
        edge_type.reshape(_NW, n_chunks, B),
    ], axis=2)
    eidx = trip.reshape(3 * E)
    return _simple_scores(x2, wcat, eidx, B=B)


# speedup vs baseline: 45.7265x; 1.0109x over previous
import functools

import jax
import jax.numpy as jnp
import numpy as np
from jax import lax
from jax.experimental import pallas as pl
from jax.experimental.pallas import tpu as pltpu
from jax.experimental.pallas import tpu_sc as plsc

_NC = 2
_NS = 16
_NW = _NC * _NS
_L = 16
_C = 128
_D = 2 * _C
_HW = _C // 2

_HI_MASK = np.uint32(0xFFFF0000)

_GATHER_DNUMS = lax.GatherDimensionNumbers(
    offset_dims=(), collapsed_slice_dims=(0,), start_index_map=(0,))


def _shuffle(v, idx):
    return lax.gather(v, idx[:, None], _GATHER_DNUMS, (1,),
                      mode=lax.GatherScatterMode.PROMISE_IN_BOUNDS)


def _hsum(v, lanes):
    for k in (8, 4, 2, 1):
        v = v + _shuffle(v, lanes ^ k)
    return v


def _sc_body(B, n_chunks, x2, wcat, eidx, out,
             idx_a, idx_b, rs_a, rd_a, rw_a, rs_b, rd_b, rw_b, ov_a, ov_b,
             semi_a, semi_b, semr_a, semr_b):
    wid = lax.axis_index("s") * _NC + lax.axis_index("c")
    ebase = wid * n_chunks * (3 * B)
    obase = wid * n_chunks * B
    lanes = lax.iota(jnp.int32, _L)

    def idx_desc(c, idxbuf, sem):
        return pltpu.make_async_copy(
            eidx.at[pl.ds(ebase + c * (3 * B), 3 * B)], idxbuf, sem)

    def row_descs(idxbuf, rs, rd, rw, sem):
        return (
            pltpu.make_async_copy(x2.at[idxbuf.at[pl.ds(0, B)]], rs, sem),
            pltpu.make_async_copy(x2.at[idxbuf.at[pl.ds(B, B)]], rd, sem),
            pltpu.make_async_copy(wcat.at[idxbuf.at[pl.ds(2 * B, B)]], rw, sem),
        )

    def fire_rows(idxbuf, rs, rd, rw, sem):
        for d in row_descs(idxbuf, rs, rd, rw, sem):
            d.start()

    def wait_rows(idxbuf, rs, rd, rw, sem):
        for d in row_descs(idxbuf, rs, rd, rw, sem):
            d.wait()

    def compute(rs, rd, rw, ov, c):
        def ext(u):
            hi = plsc.bitcast(jnp.bitwise_and(u, _HI_MASK), jnp.float32)
            lo = plsc.bitcast(jnp.left_shift(u, 16), jnp.float32)
            return hi, lo

        def edge(e, ovec):
            acc = jnp.zeros((_L,), jnp.float32)
            for j in range(_HW // _L):
                lo = j * _L
                hi = _HW + j * _L
                sa, sb = ext(rs[e, pl.ds(lo, _L)])
                wa, wb = ext(rw[e, pl.ds(lo, _L)])
                da, db = ext(rd[e, pl.ds(hi, _L)])
                acc = acc + sa * wa * da + sb * wb * db
                sa, sb = ext(rs[e, pl.ds(hi, _L)])
                wa, wb = ext(rw[e, pl.ds(hi, _L)])
                da, db = ext(rd[e, pl.ds(lo, _L)])
                acc = acc + da * wa * sa + db * wb * sb
            lane = jnp.bitwise_and(e, _L - 1)
            ovec = jnp.where(lanes == lane, _hsum(acc, lanes), ovec)

            @pl.when(lane == _L - 1)
            def _():
                base = jnp.bitwise_and(e, ~(_L - 1))
                ov[pl.ds(base, _L)] = ovec * jnp.float32(0.5)

            return ovec

        lax.fori_loop(0, B, edge, jnp.zeros((_L,), jnp.float32),
                      unroll=False)
        pltpu.sync_copy(ov, out.at[pl.ds(obase + c * B, B)])

    idx_desc(0, idx_a, semi_a).start()
    idx_desc(1, idx_b, semi_b).start()
    idx_desc(0, idx_a, semi_a).wait()
    fire_rows(idx_a, rs_a, rd_a, rw_a, semr_a)

    def step(k, carry):
        c0 = 2 * k
        idx_desc(c0 + 1, idx_b, semi_b).wait()
        fire_rows(idx_b, rs_b, rd_b, rw_b, semr_b)
        wait_rows(idx_a, rs_a, rd_a, rw_a, semr_a)
        idx_desc(c0 + 2, idx_a, semi_a).start()
        compute(rs_a, rd_a, rw_a, ov_a, c0)
        idx_desc(c0 + 2, idx_a, semi_a).wait()
        fire_rows(idx_a, rs_a, rd_a, rw_a, semr_a)
        wait_rows(idx_b, rs_b, rd_b, rw_b, semr_b)

        @pl.when(k < (n_chunks - 3) // 2)
        def _():
            idx_desc(c0 + 3, idx_b, semi_b).start()

        compute(rs_b, rd_b, rw_b, ov_b, c0 + 1)
        return carry

    lax.fori_loop(0, (n_chunks - 1) // 2, step, 0, unroll=False)

    wait_rows(idx_a, rs_a, rd_a, rw_a, semr_a)
    compute(rs_a, rd_a, rw_a, ov_a, n_chunks - 1)


@functools.partial(jax.jit, static_argnames=("B",))
def _simple_scores(x2, wcat, eidx, B=80):
    E = eidx.shape[0] // 3
    assert E % (_NW * B) == 0 and B % _L == 0
    n_chunks = E // (_NW * B)
    assert n_chunks % 2 == 1
    mesh = plsc.VectorSubcoreMesh(core_axis_name="c", subcore_axis_name="s")
    body = functools.partial(_sc_body, B, n_chunks)
    return pl.kernel(
        body,
        out_type=jax.ShapeDtypeStruct((E,), jnp.float32),
        mesh=mesh,
        compiler_params=pltpu.CompilerParams(needs_layout_passes=False),
        scratch_types=[
            pltpu.VMEM((3 * B,), jnp.int32),
            pltpu.VMEM((3 * B,), jnp.int32),
            pltpu.VMEM((B, _C), jnp.uint32),
            pltpu.VMEM((B, _C), jnp.uint32),
            pltpu.VMEM((B, _C), jnp.uint32),
            pltpu.VMEM((B, _C), jnp.uint32),
            pltpu.VMEM((B, _C), jnp.uint32),
            pltpu.VMEM((B, _C), jnp.uint32),
            pltpu.VMEM((B,), jnp.float32),
            pltpu.VMEM((B,), jnp.float32),
            pltpu.SemaphoreType.DMA,
            pltpu.SemaphoreType.DMA,
            pltpu.SemaphoreType.DMA,
            pltpu.SemaphoreType.DMA,
        ],
    )(x2, wcat, eidx)


def kernel(x, edge_index, edge_type, weights, weights_inv, B=80):
    n = x.shape[0]
    E = edge_type.shape[0]
    n_chunks = E // (_NW * B)
    x2 = lax.bitcast_convert_type(
        x.astype(jnp.bfloat16).reshape(n, _C, 2), jnp.uint32)
    wcat = lax.bitcast_convert_type(
        jnp.concatenate([weights, weights_inv], axis=1)
        .astype(jnp.bfloat16).reshape(-1, _C, 2), jnp.uint32)
    trip = jnp.stack([
        edge_index[0].reshape(_NW, n_chunks, B),
        edge_index[1].reshape(_NW, n_chunks, B),
        edge_type.reshape(_NW, n_chunks, B),
    ], axis=2)
    eidx = trip.reshape(3 * E)
    return _simple_scores(x2, wcat, eidx, B=B)


# speedup vs baseline: 57.6342x; 1.2604x over previous
import functools

import jax
import jax.numpy as jnp
import numpy as np
from jax import lax
from jax.experimental import pallas as pl
from jax.experimental.pallas import tpu as pltpu
from jax.experimental.pallas import tpu_sc as plsc

_NC = 2
_NS = 16
_NW = _NC * _NS
_L = 16
_C = 128
_D = 2 * _C
_HW = _C // 2

_HI_MASK = np.uint32(0xFFFF0000)

_GATHER_DNUMS = lax.GatherDimensionNumbers(
    offset_dims=(), collapsed_slice_dims=(0,), start_index_map=(0,))


def _shuffle(v, idx):
    return lax.gather(v, idx[:, None], _GATHER_DNUMS, (1,),
                      mode=lax.GatherScatterMode.PROMISE_IN_BOUNDS)


def _hsum(v, lanes):
    for k in (8, 4, 2, 1):
        v = v + _shuffle(v, lanes ^ k)
    return v


def _sc_body(B, n_chunks, x2, wcat, eidx, out,
             idx_a, idx_b, rs_a, rd_a, rw_a, rs_b, rd_b, rw_b, ov_a, ov_b,
             semi_a, semi_b, semr_a, semr_b):
    wid = lax.axis_index("s") * _NC + lax.axis_index("c")
    ebase = wid * n_chunks * (3 * B)
    obase = wid * n_chunks * B
    lanes = lax.iota(jnp.int32, _L)

    def idx_desc(c, idxbuf, sem):
        return pltpu.make_async_copy(
            eidx.at[pl.ds(ebase + c * (3 * B), 3 * B)], idxbuf, sem)

    def row_descs(idxbuf, rs, rd, rw, sem):
        return (
            pltpu.make_async_copy(x2.at[idxbuf.at[pl.ds(0, B)]], rs, sem),
            pltpu.make_async_copy(x2.at[idxbuf.at[pl.ds(B, B)]], rd, sem),
            pltpu.make_async_copy(wcat.at[idxbuf.at[pl.ds(2 * B, B)]], rw, sem),
        )

    def fire_rows(idxbuf, rs, rd, rw, sem):
        for d in row_descs(idxbuf, rs, rd, rw, sem):
            d.start()

    def wait_rows(idxbuf, rs, rd, rw, sem):
        for d in row_descs(idxbuf, rs, rd, rw, sem):
            d.wait()

    def compute(rs, rd, rw, ov, c):
        def ext(u):
            hi = plsc.bitcast(jnp.bitwise_and(u, _HI_MASK), jnp.float32)
            lo = plsc.bitcast(jnp.left_shift(u, 16), jnp.float32)
            return hi, lo

        def edge(e, ovec):
            acc1 = jnp.zeros((_L,), jnp.float32)
            acc2 = jnp.zeros((_L,), jnp.float32)
            acc3 = jnp.zeros((_L,), jnp.float32)
            acc4 = jnp.zeros((_L,), jnp.float32)
            for j in range(_HW // _L):
                lo = j * _L
                hi = _HW + j * _L
                sa, sb = ext(rs[e, pl.ds(lo, _L)])
                wa, wb = ext(rw[e, pl.ds(lo, _L)])
                da, db = ext(rd[e, pl.ds(hi, _L)])
                acc1 = acc1 + sa * wa * da
                acc2 = acc2 + sb * wb * db
                sa, sb = ext(rs[e, pl.ds(hi, _L)])
                wa, wb = ext(rw[e, pl.ds(hi, _L)])
                da, db = ext(rd[e, pl.ds(lo, _L)])
                acc3 = acc3 + da * wa * sa
                acc4 = acc4 + db * wb * sb
            acc = (acc1 + acc2) + (acc3 + acc4)
            lane = jnp.bitwise_and(e, _L - 1)
            ovec = jnp.where(lanes == lane, _hsum(acc, lanes), ovec)

            @pl.when(lane == _L - 1)
            def _():
                base = jnp.bitwise_and(e, ~(_L - 1))
                ov[pl.ds(base, _L)] = ovec * jnp.float32(0.5)

            return ovec

        lax.fori_loop(0, B, edge, jnp.zeros((_L,), jnp.float32),
                      unroll=4)
        pltpu.sync_copy(ov, out.at[pl.ds(obase + c * B, B)])

    idx_desc(0, idx_a, semi_a).start()
    idx_desc(1, idx_b, semi_b).start()
    idx_desc(0, idx_a, semi_a).wait()
    fire_rows(idx_a, rs_a, rd_a, rw_a, semr_a)

    def step(k, carry):
        c0 = 2 * k
        idx_desc(c0 + 1, idx_b, semi_b).wait()
        fire_rows(idx_b, rs_b, rd_b, rw_b, semr_b)
        wait_rows(idx_a, rs_a, rd_a, rw_a, semr_a)
        idx_desc(c0 + 2, idx_a, semi_a).start()
        compute(rs_a, rd_a, rw_a, ov_a, c0)
        idx_desc(c0 + 2, idx_a, semi_a).wait()
        fire_rows(idx_a, rs_a, rd_a, rw_a, semr_a)
        wait_rows(idx_b, rs_b, rd_b, rw_b, semr_b)

        @pl.when(k < (n_chunks - 3) // 2)
        def _():
            idx_desc(c0 + 3, idx_b, semi_b).start()

        compute(rs_b, rd_b, rw_b, ov_b, c0 + 1)
        return carry

    lax.fori_loop(0, (n_chunks - 1) // 2, step, 0, unroll=False)

    wait_rows(idx_a, rs_a, rd_a, rw_a, semr_a)
    compute(rs_a, rd_a, rw_a, ov_a, n_chunks - 1)


@functools.partial(jax.jit, static_argnames=("B",))
def _simple_scores(x2, wcat, eidx, B=80):
    E = eidx.shape[0] // 3
    assert E % (_NW * B) == 0 and B % _L == 0
    n_chunks = E // (_NW * B)
    assert n_chunks % 2 == 1
    mesh = plsc.VectorSubcoreMesh(core_axis_name="c", subcore_axis_name="s")
    body = functools.partial(_sc_body, B, n_chunks)
    return pl.kernel(
        body,
        out_type=jax.ShapeDtypeStruct((E,), jnp.float32),
        mesh=mesh,
        compiler_params=pltpu.CompilerParams(needs_layout_passes=False),
        scratch_types=[
            pltpu.VMEM((3 * B,), jnp.int32),
            pltpu.VMEM((3 * B,), jnp.int32),
            pltpu.VMEM((B, _C), jnp.uint32),
            pltpu.VMEM((B, _C), jnp.uint32),
            pltpu.VMEM((B, _C), jnp.uint32),
            pltpu.VMEM((B, _C), jnp.uint32),
            pltpu.VMEM((B, _C), jnp.uint32),
            pltpu.VMEM((B, _C), jnp.uint32),
            pltpu.VMEM((B,), jnp.float32),
            pltpu.VMEM((B,), jnp.float32),
            pltpu.SemaphoreType.DMA,
            pltpu.SemaphoreType.DMA,
            pltpu.SemaphoreType.DMA,
            pltpu.SemaphoreType.DMA,
        ],
    )(x2, wcat, eidx)


def kernel(x, edge_index, edge_type, weights, weights_inv, B=80):
    n = x.shape[0]
    E = edge_type.shape[0]
    n_chunks = E // (_NW * B)
    x2 = lax.bitcast_convert_type(
        x.astype(jnp.bfloat16).reshape(n, _C, 2), jnp.uint32)
    wcat = lax.bitcast_convert_type(
        jnp.concatenate([weights, weights_inv], axis=1)
        .astype(jnp.bfloat16).reshape(-1, _C, 2), jnp.uint32)
    trip = jnp.stack([
        edge_index[0].reshape(_NW, n_chunks, B),
        edge_index[1].reshape(_NW, n_chunks, B),
        edge_type.reshape(_NW, n_chunks, B),
    ], axis=2)
    eidx = trip.reshape(3 * E)
    return _simple_scores(x2, wcat, eidx, B=B)


# speedup vs baseline: 59.6782x; 1.0355x over previous
import functools

import jax
import jax.numpy as jnp
import numpy as np
from jax import lax
from jax.experimental import pallas as pl
from jax.experimental.pallas import tpu as pltpu
from jax.experimental.pallas import tpu_sc as plsc

_NC = 2
_NS = 16
_NW = _NC * _NS
_L = 16
_C = 128
_D = 2 * _C
_HW = _C // 2

_HI_MASK = np.uint32(0xFFFF0000)

_GATHER_DNUMS = lax.GatherDimensionNumbers(
    offset_dims=(), collapsed_slice_dims=(0,), start_index_map=(0,))


def _shuffle(v, idx):
    return lax.gather(v, idx[:, None], _GATHER_DNUMS, (1,),
                      mode=lax.GatherScatterMode.PROMISE_IN_BOUNDS)


def _hsum(v, lanes):
    for k in (8, 4, 2, 1):
        v = v + _shuffle(v, lanes ^ k)
    return v


def _sc_body(B, n_chunks, x2, wcat, eidx, out,
             idx_a, idx_b, rs_a, rd_a, rw_a, rs_b, rd_b, rw_b, ov_a, ov_b,
             semi_a, semi_b, semr_a, semr_b):
    wid = lax.axis_index("s") * _NC + lax.axis_index("c")
    ebase = wid * n_chunks * (3 * B)
    obase = wid * n_chunks * B
    lanes = lax.iota(jnp.int32, _L)

    def idx_desc(c, idxbuf, sem):
        return pltpu.make_async_copy(
            eidx.at[pl.ds(ebase + c * (3 * B), 3 * B)], idxbuf, sem)

    def row_descs(idxbuf, rs, rd, rw, sem):
        return (
            pltpu.make_async_copy(x2.at[idxbuf.at[pl.ds(0, B)]], rs, sem),
            pltpu.make_async_copy(x2.at[idxbuf.at[pl.ds(B, B)]], rd, sem),
            pltpu.make_async_copy(wcat.at[idxbuf.at[pl.ds(2 * B, B)]], rw, sem),
        )

    def fire_rows(idxbuf, rs, rd, rw, sem):
        for d in row_descs(idxbuf, rs, rd, rw, sem):
            d.start()

    def wait_rows(idxbuf, rs, rd, rw, sem):
        for d in row_descs(idxbuf, rs, rd, rw, sem):
            d.wait()

    def compute(rs, rd, rw, ov, c):
        def ext(u):
            hi = plsc.bitcast(jnp.bitwise_and(u, _HI_MASK), jnp.float32)
            lo = plsc.bitcast(jnp.left_shift(u, 16), jnp.float32)
            return hi, lo

        def group(g, gcarry):
            base = g * _L
            ovec = jnp.zeros((_L,), jnp.float32)
            for el in range(_L):
                e = base + el
                acc1 = jnp.zeros((_L,), jnp.float32)
                acc2 = jnp.zeros((_L,), jnp.float32)
                acc3 = jnp.zeros((_L,), jnp.float32)
                acc4 = jnp.zeros((_L,), jnp.float32)
                for j in range(_HW // _L):
                    lo = j * _L
                    hi = _HW + j * _L
                    sa, sb = ext(rs[e, pl.ds(lo, _L)])
                    wa, wb = ext(rw[e, pl.ds(lo, _L)])
                    da, db = ext(rd[e, pl.ds(hi, _L)])
                    acc1 = acc1 + sa * wa * da
                    acc2 = acc2 + sb * wb * db
                    sa, sb = ext(rs[e, pl.ds(hi, _L)])
                    wa, wb = ext(rw[e, pl.ds(hi, _L)])
                    da, db = ext(rd[e, pl.ds(lo, _L)])
                    acc3 = acc3 + da * wa * sa
                    acc4 = acc4 + db * wb * sb
                acc = (acc1 + acc2) + (acc3 + acc4)
                ovec = jnp.where(lanes == el, _hsum(acc, lanes), ovec)
            ov[pl.ds(base, _L)] = ovec * jnp.float32(0.5)
            return gcarry

        lax.fori_loop(0, B // _L, group, 0, unroll=False)
        pltpu.sync_copy(ov, out.at[pl.ds(obase + c * B, B)])

    idx_desc(0, idx_a, semi_a).start()
    idx_desc(1, idx_b, semi_b).start()
    idx_desc(0, idx_a, semi_a).wait()
    fire_rows(idx_a, rs_a, rd_a, rw_a, semr_a)

    def step(k, carry):
        c0 = 2 * k
        idx_desc(c0 + 1, idx_b, semi_b).wait()
        fire_rows(idx_b, rs_b, rd_b, rw_b, semr_b)
        wait_rows(idx_a, rs_a, rd_a, rw_a, semr_a)
        idx_desc(c0 + 2, idx_a, semi_a).start()
        compute(rs_a, rd_a, rw_a, ov_a, c0)
        idx_desc(c0 + 2, idx_a, semi_a).wait()
        fire_rows(idx_a, rs_a, rd_a, rw_a, semr_a)
        wait_rows(idx_b, rs_b, rd_b, rw_b, semr_b)

        @pl.when(k < (n_chunks - 3) // 2)
        def _():
            idx_desc(c0 + 3, idx_b, semi_b).start()

        compute(rs_b, rd_b, rw_b, ov_b, c0 + 1)
        return carry

    lax.fori_loop(0, (n_chunks - 1) // 2, step, 0, unroll=False)

    wait_rows(idx_a, rs_a, rd_a, rw_a, semr_a)
    compute(rs_a, rd_a, rw_a, ov_a, n_chunks - 1)


@functools.partial(jax.jit, static_argnames=("B",))
def _simple_scores(x2, wcat, eidx, B=80):
    E = eidx.shape[0] // 3
    assert E % (_NW * B) == 0 and B % _L == 0
    n_chunks = E // (_NW * B)
    assert n_chunks % 2 == 1
    mesh = plsc.VectorSubcoreMesh(core_axis_name="c", subcore_axis_name="s")
    body = functools.partial(_sc_body, B, n_chunks)
    return pl.kernel(
        body,
        out_type=jax.ShapeDtypeStruct((E,), jnp.float32),
        mesh=mesh,
        compiler_params=pltpu.CompilerParams(needs_layout_passes=False),
        scratch_types=[
            pltpu.VMEM((3 * B,), jnp.int32),
            pltpu.VMEM((3 * B,), jnp.int32),
            pltpu.VMEM((B, _C), jnp.uint32),
            pltpu.VMEM((B, _C), jnp.uint32),
            pltpu.VMEM((B, _C), jnp.uint32),
            pltpu.VMEM((B, _C), jnp.uint32),
            pltpu.VMEM((B, _C), jnp.uint32),
            pltpu.VMEM((B, _C), jnp.uint32),
            pltpu.VMEM((B,), jnp.float32),
            pltpu.VMEM((B,), jnp.float32),
            pltpu.SemaphoreType.DMA,
            pltpu.SemaphoreType.DMA,
            pltpu.SemaphoreType.DMA,
            pltpu.SemaphoreType.DMA,
        ],
    )(x2, wcat, eidx)


def kernel(x, edge_index, edge_type, weights, weights_inv, B=80):
    n = x.shape[0]
    E = edge_type.shape[0]
    n_chunks = E // (_NW * B)
    x2 = lax.bitcast_convert_type(
        x.astype(jnp.bfloat16).reshape(n, _C, 2), jnp.uint32)
    wcat = lax.bitcast_convert_type(
        jnp.concatenate([weights, weights_inv], axis=1)
        .astype(jnp.bfloat16).reshape(-1, _C, 2), jnp.uint32)
    trip = jnp.stack([
        edge_index[0].reshape(_NW, n_chunks, B),
        edge_index[1].reshape(_NW, n_chunks, B),
        edge_type.reshape(_NW, n_chunks, B),
    ], axis=2)
    eidx = trip.reshape(3 * E)
    return _simple_scores(x2, wcat, eidx, B=B)


# speedup vs baseline: 68.0720x; 1.1407x over previous
import functools

import jax
import jax.numpy as jnp
import numpy as np
from jax import lax
from jax.experimental import pallas as pl
from jax.experimental.pallas import tpu as pltpu
from jax.experimental.pallas import tpu_sc as plsc

_NC = 2
_NS = 16
_NW = _NC * _NS
_L = 16
_C = 128
_D = 2 * _C
_HW = _C // 2

_HI_MASK = np.uint32(0xFFFF0000)

_GATHER_DNUMS = lax.GatherDimensionNumbers(
    offset_dims=(), collapsed_slice_dims=(0,), start_index_map=(0,))


def _shuffle(v, idx):
    return lax.gather(v, idx[:, None], _GATHER_DNUMS, (1,),
                      mode=lax.GatherScatterMode.PROMISE_IN_BOUNDS)


def _hsum(v, lanes):
    for k in (8, 4, 2, 1):
        v = v + _shuffle(v, lanes ^ k)
    return v


def _pack_rows(a):
    u = lax.bitcast_convert_type(a.astype(jnp.bfloat16), jnp.uint16)
    u3 = u.reshape(a.shape[0], -1, 2).astype(jnp.uint32)
    return jnp.bitwise_or(jnp.left_shift(u3[:, :, 1], 16), u3[:, :, 0])


def _sc_body(B, n_chunks, x2, wcat, src, dst, et, out,
             idx_a, idx_b, rs_a, rd_a, rw_a, rs_b, rd_b, rw_b, ov_a, ov_b,
             semi_a, semi_b, semr_a, semr_b):
    epw = n_chunks * B
    wid = lax.axis_index("s") * _NC + lax.axis_index("c")
    base = wid * epw
    lanes = lax.iota(jnp.int32, _L)

    def idx_descs(c, idxbuf, sem):
        off = base + c * B
        return (
            pltpu.make_async_copy(src.at[pl.ds(off, B)], idxbuf.at[0], sem),
            pltpu.make_async_copy(dst.at[pl.ds(off, B)], idxbuf.at[1], sem),
            pltpu.make_async_copy(et.at[pl.ds(off, B)], idxbuf.at[2], sem),
        )

    def row_descs(idxbuf, rs, rd, rw, sem):
        return (
            pltpu.make_async_copy(x2.at[idxbuf.at[0]], rs, sem),
            pltpu.make_async_copy(x2.at[idxbuf.at[1]], rd, sem),
            pltpu.make_async_copy(wcat.at[idxbuf.at[2]], rw, sem),
        )

    def fire(descs):
        for d in descs:
            d.start()

    def wait(descs):
        for d in descs:
            d.wait()

    def compute(rs, rd, rw, ov, c):
        def ext(u):
            hi = plsc.bitcast(jnp.bitwise_and(u, _HI_MASK), jnp.float32)
            lo = plsc.bitcast(jnp.left_shift(u, 16), jnp.float32)
            return hi, lo

        def group(g, gcarry):
            gbase = g * _L
            ovec = jnp.zeros((_L,), jnp.float32)
            for el in range(_L):
                e = gbase + el
                acc1 = jnp.zeros((_L,), jnp.float32)
                acc2 = jnp.zeros((_L,), jnp.float32)
                acc3 = jnp.zeros((_L,), jnp.float32)
                acc4 = jnp.zeros((_L,), jnp.float32)
                for j in range(_HW // _L):
                    lo = j * _L
                    hi = _HW + j * _L
                    sa, sb = ext(rs[e, pl.ds(lo, _L)])
                    wa, wb = ext(rw[e, pl.ds(lo, _L)])
                    da, db = ext(rd[e, pl.ds(hi, _L)])
                    acc1 = acc1 + sa * wa * da
                    acc2 = acc2 + sb * wb * db
                    sa, sb = ext(rs[e, pl.ds(hi, _L)])
                    wa, wb = ext(rw[e, pl.ds(hi, _L)])
                    da, db = ext(rd[e, pl.ds(lo, _L)])
                    acc3 = acc3 + da * wa * sa
                    acc4 = acc4 + db * wb * sb
                acc = (acc1 + acc2) + (acc3 + acc4)
                ovec = jnp.where(lanes == el, _hsum(acc, lanes), ovec)
            ov[pl.ds(gbase, _L)] = ovec * jnp.float32(0.5)
            return gcarry

        lax.fori_loop(0, B // _L, group, 0, unroll=False)
        pltpu.sync_copy(ov, out.at[pl.ds(base + c * B, B)])

    fire(idx_descs(0, idx_a, semi_a))
    fire(idx_descs(1, idx_b, semi_b))
    wait(idx_descs(0, idx_a, semi_a))
    fire(row_descs(idx_a, rs_a, rd_a, rw_a, semr_a))

    def step(k, carry):
        c0 = 2 * k
        wait(idx_descs(c0 + 1, idx_b, semi_b))
        fire(row_descs(idx_b, rs_b, rd_b, rw_b, semr_b))
        wait(row_descs(idx_a, rs_a, rd_a, rw_a, semr_a))
        fire(idx_descs(c0 + 2, idx_a, semi_a))
        compute(rs_a, rd_a, rw_a, ov_a, c0)
        wait(idx_descs(c0 + 2, idx_a, semi_a))
        fire(row_descs(idx_a, rs_a, rd_a, rw_a, semr_a))
        wait(row_descs(idx_b, rs_b, rd_b, rw_b, semr_b))

        @pl.when(k < (n_chunks - 3) // 2)
        def _():
            fire(idx_descs(c0 + 3, idx_b, semi_b))

        compute(rs_b, rd_b, rw_b, ov_b, c0 + 1)
        return carry

    lax.fori_loop(0, (n_chunks - 1) // 2, step, 0, unroll=False)

    wait(row_descs(idx_a, rs_a, rd_a, rw_a, semr_a))
    compute(rs_a, rd_a, rw_a, ov_a, n_chunks - 1)


@functools.partial(jax.jit, static_argnames=("B",))
def _simple_scores(x2, wcat, src, dst, et, B=80):
    E = src.shape[0]
    assert E % (_NW * B) == 0 and B % _L == 0
    n_chunks = E // (_NW * B)
    assert n_chunks % 2 == 1
    mesh = plsc.VectorSubcoreMesh(core_axis_name="c", subcore_axis_name="s")
    body = functools.partial(_sc_body, B, n_chunks)
    return pl.kernel(
        body,
        out_type=jax.ShapeDtypeStruct((E,), jnp.float32),
        mesh=mesh,
        compiler_params=pltpu.CompilerParams(needs_layout_passes=False),
        scratch_types=[
            pltpu.VMEM((3, B), jnp.int32),
            pltpu.VMEM((3, B), jnp.int32),
            pltpu.VMEM((B, _C), jnp.uint32),
            pltpu.VMEM((B, _C), jnp.uint32),
            pltpu.VMEM((B, _C), jnp.uint32),
            pltpu.VMEM((B, _C), jnp.uint32),
            pltpu.VMEM((B, _C), jnp.uint32),
            pltpu.VMEM((B, _C), jnp.uint32),
            pltpu.VMEM((B,), jnp.float32),
            pltpu.VMEM((B,), jnp.float32),
            pltpu.SemaphoreType.DMA,
            pltpu.SemaphoreType.DMA,
            pltpu.SemaphoreType.DMA,
            pltpu.SemaphoreType.DMA,
        ],
    )(x2, wcat, src, dst, et)


def kernel(x, edge_index, edge_type, weights, weights_inv, B=80):
    n = x.shape[0]
    x2 = _pack_rows(x.reshape(n, _D))
    wcat = _pack_rows(jnp.concatenate([weights, weights_inv], axis=1))
    return _simple_scores(x2, wcat, edge_index[0], edge_index[1], edge_type,
                          B=B)


# speedup vs baseline: 79.5635x; 1.1688x over previous
import functools

import jax
import jax.numpy as jnp
import numpy as np
from jax import lax
from jax.experimental import pallas as pl
from jax.experimental.pallas import tpu as pltpu
from jax.experimental.pallas import tpu_sc as plsc

_NC = 2
_NS = 16
_NW = _NC * _NS
_L = 16
_C = 128
_D = 2 * _C
_HW = _C // 2

_HI_MASK = np.uint32(0xFFFF0000)

_GATHER_DNUMS = lax.GatherDimensionNumbers(
    offset_dims=(), collapsed_slice_dims=(0,), start_index_map=(0,))


def _shuffle(v, idx):
    return lax.gather(v, idx[:, None], _GATHER_DNUMS, (1,),
                      mode=lax.GatherScatterMode.PROMISE_IN_BOUNDS)


def _hsum(v, lanes):
    for k in (8, 4, 2, 1):
        v = v + _shuffle(v, lanes ^ k)
    return v


def _pack_rows(a):
    v = lax.bitcast_convert_type(a, jnp.uint32)
    r = v + np.uint32(0x7FFF) + jnp.bitwise_and(
        jnp.right_shift(v, 16), np.uint32(1))
    hi = jnp.bitwise_and(r, _HI_MASK)
    lo = jnp.right_shift(r, 16)
    h = jnp.bitwise_or(hi[:, _HW:_C], lo[:, 0:_HW])
    t = jnp.bitwise_or(hi[:, _C + _HW:_D], lo[:, _C:_C + _HW])
    return jnp.concatenate([h, t], axis=1)


def _sc_body(B, n_chunks, x2, wcat, src, dst, et, out,
             ixs, ixd, ixt, rs_a, rd_a, rw_a, rs_b, rd_b, rw_b, ov_a, ov_b,
             semi, semr_a, semr_b, semo_a, semo_b):
    epw = n_chunks * B
    wid = lax.axis_index("s") * _NC + lax.axis_index("c")
    base = wid * epw
    lanes = lax.iota(jnp.int32, _L)

    def idx_descs():
        return (
            pltpu.make_async_copy(src.at[pl.ds(base, epw)], ixs, semi),
            pltpu.make_async_copy(dst.at[pl.ds(base, epw)], ixd, semi),
            pltpu.make_async_copy(et.at[pl.ds(base, epw)], ixt, semi),
        )

    def row_descs(c, rs, rd, rw, sem):
        off = c * B
        return (
            pltpu.make_async_copy(x2.at[ixs.at[pl.ds(off, B)]], rs, sem),
            pltpu.make_async_copy(x2.at[ixd.at[pl.ds(off, B)]], rd, sem),
            pltpu.make_async_copy(wcat.at[ixt.at[pl.ds(off, B)]], rw, sem),
        )

    def out_desc(c, ov, sem):
        return pltpu.make_async_copy(ov, out.at[pl.ds(base + c * B, B)], sem)

    def fire(descs):
        for d in descs:
            d.start()

    def wait(descs):
        for d in descs:
            d.wait()

    def compute(rs, rd, rw, ov):
        def ext(u):
            hi = plsc.bitcast(jnp.bitwise_and(u, _HI_MASK), jnp.float32)
            lo = plsc.bitcast(jnp.left_shift(u, 16), jnp.float32)
            return hi, lo

        def group(g, gcarry):
            gbase = g * _L
            ovec = jnp.zeros((_L,), jnp.float32)
            for el in range(_L):
                e = gbase + el
                acc1 = jnp.zeros((_L,), jnp.float32)
                acc2 = jnp.zeros((_L,), jnp.float32)
                acc3 = jnp.zeros((_L,), jnp.float32)
                acc4 = jnp.zeros((_L,), jnp.float32)
                for j in range(_HW // _L):
                    lo = j * _L
                    hi = _HW + j * _L
                    sa, sb = ext(rs[e, pl.ds(lo, _L)])
                    wa, wb = ext(rw[e, pl.ds(lo, _L)])
                    da, db = ext(rd[e, pl.ds(hi, _L)])
                    acc1 = acc1 + sa * wa * da
                    acc2 = acc2 + sb * wb * db
                    sa, sb = ext(rs[e, pl.ds(hi, _L)])
                    wa, wb = ext(rw[e, pl.ds(hi, _L)])
                    da, db = ext(rd[e, pl.ds(lo, _L)])
                    acc3 = acc3 + da * wa * sa
                    acc4 = acc4 + db * wb * sb
                acc = (acc1 + acc2) + (acc3 + acc4)
                ovec = jnp.where(lanes == el, _hsum(acc, lanes), ovec)
            ov[pl.ds(gbase, _L)] = ovec * jnp.float32(0.5)
            return gcarry

        lax.fori_loop(0, B // _L, group, 0, unroll=False)

    fire(idx_descs())
    wait(idx_descs())
    fire(row_descs(0, rs_a, rd_a, rw_a, semr_a))

    def step(k, carry):
        c0 = 2 * k
        fire(row_descs(c0 + 1, rs_b, rd_b, rw_b, semr_b))
        wait(row_descs(c0, rs_a, rd_a, rw_a, semr_a))

        @pl.when(k > 0)
        def _():
            out_desc(c0 - 2, ov_a, semo_a).wait()

        compute(rs_a, rd_a, rw_a, ov_a)
        out_desc(c0, ov_a, semo_a).start()
        fire(row_descs(c0 + 2, rs_a, rd_a, rw_a, semr_a))
        wait(row_descs(c0 + 1, rs_b, rd_b, rw_b, semr_b))

        @pl.when(k > 0)
        def _():
            out_desc(c0 - 1, ov_b, semo_b).wait()

        compute(rs_b, rd_b, rw_b, ov_b)
        out_desc(c0 + 1, ov_b, semo_b).start()
        return carry

    n_steps = (n_chunks - 1) // 2
    lax.fori_loop(0, n_steps, step, 0, unroll=False)

    last = n_chunks - 1
    wait(row_descs(last, rs_a, rd_a, rw_a, semr_a))
    out_desc(last - 2, ov_a, semo_a).wait()
    compute(rs_a, rd_a, rw_a, ov_a)
    out_desc(last, ov_a, semo_a).start()
    out_desc(last, ov_a, semo_a).wait()
    out_desc(last - 1, ov_b, semo_b).wait()


@functools.partial(jax.jit, static_argnames=("B",))
def _simple_scores(x2, wcat, src, dst, et, B=80):
    E = src.shape[0]
    assert E % (_NW * B) == 0 and B % _L == 0
    n_chunks = E // (_NW * B)
    assert n_chunks % 2 == 1 and n_chunks >= 3
    mesh = plsc.VectorSubcoreMesh(core_axis_name="c", subcore_axis_name="s")
    body = functools.partial(_sc_body, B, n_chunks)
    return pl.kernel(
        body,
        out_type=jax.ShapeDtypeStruct((E,), jnp.float32),
        mesh=mesh,
        compiler_params=pltpu.CompilerParams(needs_layout_passes=False),
        scratch_types=[
            pltpu.VMEM((n_chunks * B,), jnp.int32),
            pltpu.VMEM((n_chunks * B,), jnp.int32),
            pltpu.VMEM((n_chunks * B,), jnp.int32),
            pltpu.VMEM((B, _C), jnp.uint32),
            pltpu.VMEM((B, _C), jnp.uint32),
            pltpu.VMEM((B, _C), jnp.uint32),
            pltpu.VMEM((B, _C), jnp.uint32),
            pltpu.VMEM((B, _C), jnp.uint32),
            pltpu.VMEM((B, _C), jnp.uint32),
            pltpu.VMEM((B,), jnp.float32),
            pltpu.VMEM((B,), jnp.float32),
            pltpu.SemaphoreType.DMA,
            pltpu.SemaphoreType.DMA,
            pltpu.SemaphoreType.DMA,
            pltpu.SemaphoreType.DMA,
            pltpu.SemaphoreType.DMA,
        ],
    )(x2, wcat, src, dst, et)


def kernel(x, edge_index, edge_type, weights, weights_inv, B=80):
    n = x.shape[0]
    x2 = _pack_rows(x.reshape(n, _D))
    wcat = _pack_rows(jnp.concatenate([weights, weights_inv], axis=1))
    return _simple_scores(x2, wcat, edge_index[0], edge_index[1], edge_type,
                          B=B)
